# Initial kernel scaffold; baseline (speedup 1.0000x reference)
import jax
import jax.numpy as jnp
from jax.experimental import pallas as pl


def kernel(user_id, history, top_genres, item_id, tmdb_genres, user_avg_rating, activity, release_year, item_avg_rating, revenue, UT_user_emb, UT_item_emb, UT_genre_emb, UT_cont_W, UT_cont_b, UT_W1, UT_b1, UT_W2, UT_b2, IT_item_emb, IT_genre_emb, IT_cont_W, IT_cont_b, IT_W1, IT_b1, IT_W2, IT_b2):
    raise NotImplementedError("write your pallas kernel here")


# speedup vs baseline: 1.7671x; 1.7671x over previous
import functools

import jax
import jax.numpy as jnp
from jax import lax
from jax.experimental import pallas as pl
from jax.experimental.pallas import tpu as pltpu
from jax.experimental.pallas import tpu_sc as plsc

B = 4096
D = 64
L = 200
LP = 208
LH = 104
G = 5
GP = 8
NC = 2
NS = 16
NW = NC * NS
BPW = B // NW
HROWS = 8
NCHUNK = BPW // HROWS

_f32 = jnp.float32
_i32 = jnp.int32


def _sum_rows(buf, start, n, unroll):
    def body(l, acc):
        r = start + l
        return tuple(acc[c] + buf[r, pl.ds(c * 16, 16)] for c in range(4))
    init = tuple(jnp.zeros((16,), _f32) for _ in range(4))
    return lax.fori_loop(0, n, body, init, unroll=unroll)


def _sc_pool_body(hist_hbm, tg_hbm, ig_hbm, uid_hbm, iid_hbm,
                  ut_item, ut_genre, it_genre, ut_user, it_item,
                  hist_sum, ug_sum, ig_sum, u_rows, i_rows,
                  idx128_v, rows128_v, gacc_v, hidx_v, hbufa_v, hbufb_v,
                  hstage_v, sem0, sem1, sem2):
    wid = lax.axis_index("s") * NC + lax.axis_index("c")
    base = wid * BPW

    pltpu.sync_copy(uid_hbm.at[pl.ds(base, BPW)], idx128_v)
    pltpu.async_copy(ut_user.at[idx128_v], rows128_v, sem2).wait()
    pltpu.sync_copy(rows128_v, u_rows.at[pl.ds(base, BPW)])

    pltpu.sync_copy(iid_hbm.at[pl.ds(base, BPW)], idx128_v)
    pltpu.async_copy(it_item.at[idx128_v], rows128_v, sem2).wait()
    pltpu.sync_copy(rows128_v, i_rows.at[pl.ds(base, BPW)])

    for tbl, src, dst in ((ut_genre, tg_hbm, ug_sum), (it_genre, ig_hbm, ig_sum)):
        for k in range(BPW * GP // 128):
            pltpu.sync_copy(src.at[pl.ds(base * GP + k * 128, 128)], idx128_v)
            pltpu.async_copy(tbl.at[idx128_v], rows128_v, sem2).wait()

            @pl.loop(0, 16)
            def _(r):
                acc = _sum_rows(rows128_v, r * GP, GP, unroll=8)
                for c in range(4):
                    gacc_v[r, pl.ds(c * 16, 16)] = acc[c]

            pltpu.sync_copy(gacc_v, dst.at[pl.ds(base + k * 16, 16)])

    hbufs = (hbufa_v, hbufb_v)
    hsems = (sem0, sem1)

    @pl.loop(0, NCHUNK)
    def _(ci):
        rbase = base + ci * HROWS
        pltpu.sync_copy(hist_hbm.at[pl.ds(rbase * LP, HROWS * LP)], hidx_v)

        def issue(j, bi):
            buf, sem = hbufs[bi], hsems[bi]
            h1 = pltpu.async_copy(
                ut_item.at[hidx_v.at[pl.ds(j * LP, LH)]],
                buf.at[pl.ds(0, LH)], sem)
            h2 = pltpu.async_copy(
                ut_item.at[hidx_v.at[pl.ds(j * LP + LH, LH)]],
                buf.at[pl.ds(LH, LH)], sem)
            return h1, h2

        pend = issue(0, 0)
        for j in range(HROWS):
            bi = j % 2
            cur = pend
            if j + 1 < HROWS:
                pend = issue(j + 1, 1 - bi)
            cur[0].wait()
            cur[1].wait()
            acc = _sum_rows(hbufs[bi], 0, LP, unroll=8)
            for c in range(4):
                hstage_v[j, pl.ds(c * 16, 16)] = acc[c]
        pltpu.sync_copy(hstage_v, hist_sum.at[pl.ds(rbase, HROWS)])


@functools.cache
def _sc_pool_kernel():
  return pl.kernel(
    _sc_pool_body,
    out_type=[jax.ShapeDtypeStruct((B, D), _f32) for _ in range(5)],
    mesh=plsc.VectorSubcoreMesh(core_axis_name="c", subcore_axis_name="s"),
    compiler_params=pltpu.CompilerParams(use_tc_tiling_on_sc=False),
    scratch_types=[
        pltpu.VMEM((128,), _i32),
        pltpu.VMEM((128, D), _f32),
        pltpu.VMEM((16, D), _f32),
        pltpu.VMEM((HROWS * LP,), _i32),
        pltpu.VMEM((LP, D), _f32),
        pltpu.VMEM((LP, D), _f32),
        pltpu.VMEM((HROWS, D), _f32),
        pltpu.SemaphoreType.DMA,
        pltpu.SemaphoreType.DMA,
        pltpu.SemaphoreType.DMA,
    ],
  )


def _towers_body(hs, ugs, igs, ur, ir, hidx, tgidx, igidx, uid, iid, uc, ic,
                 row0, ucw, ucb, uw1, ub1, uw2, ub2,
                 icw, icb, iw1, ib1, iw2, ib2, uout, iout):
    hp = jax.lax.Precision.HIGHEST

    zh = jnp.sum((hidx[...] == 0).astype(_f32), axis=1, keepdims=True)
    hist_mean = (hs[...] - zh * row0[0:1, :]) / ((LP - zh) + 1e-8)
    ztg = jnp.sum((tgidx[...] == 0).astype(_f32), axis=1, keepdims=True)
    ug_mean = (ugs[...] - ztg * row0[1:2, :]) / ((GP - ztg) + 1e-8)
    zig = jnp.sum((igidx[...] == 0).astype(_f32), axis=1, keepdims=True)
    ig_mean = (igs[...] - zig * row0[2:3, :]) / ((GP - zig) + 1e-8)

    u_emb = ur[...] * (uid[...] > 0).astype(_f32)
    i_emb = ir[...] * (iid[...] > 0).astype(_f32)

    ucv = uc[...]
    u_cont = jnp.maximum(
        ucv[:, 0:1] * ucw[0:1, :] + ucv[:, 1:2] * ucw[1:2, :] + ucb[...], 0.0)
    icv = ic[...]
    i_cont = jnp.maximum(
        icv[:, 0:1] * icw[0:1, :] + icv[:, 1:2] * icw[1:2, :]
        + icv[:, 2:3] * icw[2:3, :] + icb[...], 0.0)

    u_cat = jnp.concatenate([u_emb, hist_mean, ug_mean, u_cont], axis=1)
    h1 = jnp.maximum(
        jnp.dot(u_cat, uw1[...], preferred_element_type=_f32, precision=hp)
        + ub1[...], 0.0)
    u2 = jnp.dot(h1, uw2[...], preferred_element_type=_f32, precision=hp) + ub2[...]
    un = jnp.sqrt(jnp.sum(u2 * u2, axis=1, keepdims=True))
    uout[...] = u2 / jnp.maximum(un, 1e-12)

    i_cat = jnp.concatenate([i_emb, ig_mean, i_cont], axis=1)
    g1 = jnp.maximum(
        jnp.dot(i_cat, iw1[...], preferred_element_type=_f32, precision=hp)
        + ib1[...], 0.0)
    i2 = jnp.dot(g1, iw2[...], preferred_element_type=_f32, precision=hp) + ib2[...]
    inn = jnp.sqrt(jnp.sum(i2 * i2, axis=1, keepdims=True))
    iout[...] = i2 / jnp.maximum(inn, 1e-12)


_BB = 512


def _tc_block_specs():
    def blk(n, dt=None):
        return pl.BlockSpec((_BB, n), lambda i: (i, 0))

    def rep(shape):
        return pl.BlockSpec(shape, lambda i: (0, 0))

    in_specs = [
        blk(D), blk(D), blk(D), blk(D), blk(D),
        blk(LP), blk(GP), blk(GP), blk(1), blk(1),
        blk(2), blk(3),
        rep((3, D)),
        rep((2, D)), rep((1, D)),
        rep((4 * D, 128)), rep((1, 128)),
        rep((128, D)), rep((1, D)),
        rep((3, D)), rep((1, D)),
        rep((3 * D, 128)), rep((1, 128)),
        rep((128, D)), rep((1, D)),
    ]
    out_specs = [blk(D), blk(D)]
    return in_specs, out_specs


def _tc_towers(*args):
    in_specs, out_specs = _tc_block_specs()
    return pl.pallas_call(
        _towers_body,
        grid=(B // _BB,),
        in_specs=in_specs,
        out_specs=out_specs,
        out_shape=[jax.ShapeDtypeStruct((B, D), _f32) for _ in range(2)],
    )(*args)


def kernel(user_id, history, top_genres, item_id, tmdb_genres,
           user_avg_rating, activity, release_year, item_avg_rating, revenue,
           UT_user_emb, UT_item_emb, UT_genre_emb, UT_cont_W, UT_cont_b,
           UT_W1, UT_b1, UT_W2, UT_b2,
           IT_item_emb, IT_genre_emb, IT_cont_W, IT_cont_b,
           IT_W1, IT_b1, IT_W2, IT_b2):
    hist = jnp.concatenate(
        [history.astype(_i32), jnp.zeros((B, LP - L), _i32)], axis=1)
    tg = jnp.concatenate(
        [top_genres.astype(_i32), jnp.zeros((B, GP - G), _i32)], axis=1)
    ig = jnp.concatenate(
        [tmdb_genres.astype(_i32), jnp.zeros((B, GP - G), _i32)], axis=1)
    uid = user_id.astype(_i32)
    iid = item_id.astype(_i32)

    hist_sum, ug_sum, ig_sum, u_rows, i_rows = _sc_pool_kernel()(
        hist.reshape(-1), tg.reshape(-1), ig.reshape(-1), uid, iid,
        UT_item_emb, UT_genre_emb, IT_genre_emb, UT_user_emb, IT_item_emb)

    row0 = jnp.stack([UT_item_emb[0], UT_genre_emb[0], IT_genre_emb[0]], axis=0)
    ucont = jnp.stack([user_avg_rating, activity], axis=1).astype(_f32)
    icont = jnp.stack(
        [release_year, item_avg_rating, revenue], axis=1).astype(_f32)

    user_out, item_out = _tc_towers(
        hist_sum, ug_sum, ig_sum, u_rows, i_rows,
        hist, tg, ig, uid[:, None], iid[:, None], ucont, icont,
        row0,
        UT_cont_W.T, UT_cont_b[None, :],
        UT_W1.T, UT_b1[None, :], UT_W2.T, UT_b2[None, :],
        IT_cont_W.T, IT_cont_b[None, :],
        IT_W1.T, IT_b1[None, :], IT_W2.T, IT_b2[None, :])
    return user_out, item_out
